# Initial kernel scaffold; baseline (speedup 1.0000x reference)
import jax
import jax.numpy as jnp
from jax.experimental import pallas as pl


def kernel(user_emb, item_emb, train_user, train_item):
    raise NotImplementedError("write your pallas kernel here")


# speedup vs baseline: 33.3840x; 33.3840x over previous
import functools

import jax
import jax.numpy as jnp
from jax import lax
from jax.experimental import pallas as pl
from jax.experimental.pallas import tpu as pltpu
from jax.experimental.pallas import tpu_sc as plsc

N_U = 50000
N_I = 50000
NP = 51200
D = 32
E = 1600000
NS = 16
SPAN = NP // NS
RCH = 160
NRCH = SPAN // RCH
R2 = 80
NRCH2 = SPAN // R2
H = 16
SL = 125
ROWS_TOT = E // SL
RPT = ROWS_TOT // NS
NB = 2
NBLK = RPT // NB
NPAIR = NBLK // 2
NB_D = 8
NBLK_D = RPT // NB_D
NPAIR_D = NBLK_D // 2

_mesh = plsc.VectorSubcoreMesh(
    core_axis_name="c", subcore_axis_name="s", num_cores=2, num_subcores=NS)


def _f32(shape):
    return jax.ShapeDtypeStruct(shape, jnp.float32)


def _zero_rows(rbuf, nrows):
    z = jnp.zeros((H,), jnp.float32)

    def st(r, carry):
        rbuf[r, pl.ds(0, H)] = z
        rbuf[r, pl.ds(H, H)] = z
        return carry

    lax.fori_loop(0, nrows, st, 0)


def _scale_rows(rbuf, wbuf, koff, nrows=RCH):
    def st(t, carry):
        wvec = wbuf[pl.ds(koff + t * H, H)]
        for ri in range(H):
            r = t * H + ri
            w = wvec[ri]
            rbuf[r, pl.ds(0, H)] = rbuf[r, pl.ds(0, H)] * w
            rbuf[r, pl.ds(H, H)] = rbuf[r, pl.ds(H, H)] * w
        return carry

    lax.fori_loop(0, nrows // H, st, 0)


def _edge_pipeline(tab, acc, sidx_hbm, didx_hbm, sA, sB, dA, dB, rA, rB,
                   gsA, gsB, ssA, ssB, row_lo):
    def load_idx(b, sidx, didx):
        r0 = row_lo + b * NB
        pltpu.sync_copy(sidx_hbm.at[pl.ds(r0, NB), :], sidx)
        pltpu.sync_copy(didx_hbm.at[pl.ds(r0, NB), :], didx)

    def fire_g(sidx, rows, gsem):
        for j in range(NB):
            pltpu.async_copy(tab.at[sidx.at[j]], rows.at[j], gsem)

    def wait_g(sidx, rows, gsem):
        for j in range(NB):
            pltpu.make_async_copy(tab.at[sidx.at[j]], rows.at[j], gsem).wait()

    def fire_s(didx, rows, ssem):
        for j in range(NB):
            pltpu.async_copy(rows.at[j], acc.at[didx.at[j]], ssem, add=True)

    def wait_s(didx, rows, ssem):
        for j in range(NB):
            pltpu.make_async_copy(rows.at[j], acc.at[didx.at[j]], ssem).wait()

    def pair(g, carry):
        ge1 = g >= 1

        @pl.when(ge1)
        def _():
            wait_s(dA, rA, ssA)

        load_idx(2 * g, sA, dA)
        fire_g(sA, rA, gsA)

        @pl.when(ge1)
        def _():
            wait_g(sB, rB, gsB)
            fire_s(dB, rB, ssB)
            wait_s(dB, rB, ssB)

        load_idx(2 * g + 1, sB, dB)
        fire_g(sB, rB, gsB)

        wait_g(sA, rA, gsA)
        fire_s(dA, rA, ssA)
        return carry

    lax.fori_loop(0, NPAIR, pair, 0)
    wait_g(sB, rB, gsB)
    fire_s(dB, rB, ssB)
    wait_s(dA, rA, ssA)
    wait_s(dB, rB, ssB)


def _deg_pipeline(acc1, didx_hbm, dA, dB, ones_v, ssA, ssB, row_lo):
    def load_idx(b, didx):
        r0 = row_lo + b * NB_D
        pltpu.sync_copy(didx_hbm.at[pl.ds(r0, NB_D), :], didx)

    def fire_s(didx, ssem):
        for j in range(NB_D):
            pltpu.async_copy(ones_v, acc1.at[didx.at[j]], ssem, add=True)

    def wait_s(didx, ssem):
        for j in range(NB_D):
            pltpu.make_async_copy(ones_v, acc1.at[didx.at[j]], ssem).wait()

    def pair(g, carry):
        ge1 = g >= 1

        @pl.when(ge1)
        def _():
            wait_s(dA, ssA)

        load_idx(2 * g, dA)
        fire_s(dA, ssA)

        @pl.when(ge1)
        def _():
            wait_s(dB, ssB)

        load_idx(2 * g + 1, dB)
        fire_s(dB, ssB)
        return carry

    lax.fori_loop(0, NPAIR_D, pair, 0)
    wait_s(dA, ssA)
    wait_s(dB, ssB)


@functools.partial(
    pl.kernel,
    out_type=(_f32((NP,)), _f32((NP,))),
    mesh=_mesh,
    compiler_params=pltpu.CompilerParams(use_tc_tiling_on_sc=False),
    scratch_types=[
        pltpu.VMEM((NB_D, SL), jnp.int32),
        pltpu.VMEM((NB_D, SL), jnp.int32),
        pltpu.VMEM((SL,), jnp.float32),
        pltpu.VMEM((SPAN,), jnp.float32),
        pltpu.VMEM_SHARED((NP,), jnp.float32),
        pltpu.SemaphoreType.DMA,
        pltpu.SemaphoreType.DMA,
    ],
)
def _deg_kernel(tu_hbm, ti_hbm, ones_hbm, degu, degi,
                dA, dB, ones_v, sbuf, acc1, ssA, ssB):
    c = lax.axis_index("c")
    s = lax.axis_index("s")
    row0 = s * SPAN

    z = jnp.zeros((H,), jnp.float32)

    def zst(r, carry):
        sbuf[pl.ds(r * H, H)] = z
        return carry

    lax.fori_loop(0, SPAN // H, zst, 0)
    pltpu.sync_copy(sbuf, acc1.at[pl.ds(row0, SPAN)])
    pltpu.sync_copy(ones_hbm, ones_v)
    plsc.subcore_barrier()

    row_lo = s * RPT

    @pl.when(c == 0)
    def _():
        _deg_pipeline(acc1, tu_hbm, dA, dB, ones_v, ssA, ssB, row_lo)

    @pl.when(c == 1)
    def _():
        _deg_pipeline(acc1, ti_hbm, dA, dB, ones_v, ssA, ssB, row_lo)

    plsc.subcore_barrier()

    pltpu.sync_copy(acc1.at[pl.ds(row0, SPAN)], sbuf)

    @pl.when(c == 0)
    def _():
        pltpu.sync_copy(sbuf, degu.at[pl.ds(row0, SPAN)])

    @pl.when(c == 1)
    def _():
        pltpu.sync_copy(sbuf, degi.at[pl.ds(row0, SPAN)])


_LAYER_SCRATCH = [
    pltpu.VMEM((NB, SL), jnp.int32),
    pltpu.VMEM((NB, SL), jnp.int32),
    pltpu.VMEM((NB, SL), jnp.int32),
    pltpu.VMEM((NB, SL), jnp.int32),
    pltpu.VMEM((NB, SL, D), jnp.float32),
    pltpu.VMEM((NB, SL, D), jnp.float32),
    pltpu.VMEM((SPAN,), jnp.float32),
    pltpu.VMEM((RCH, D), jnp.float32),
    pltpu.VMEM_SHARED((NP, D), jnp.float32),
    pltpu.SemaphoreType.DMA,
    pltpu.SemaphoreType.DMA,
    pltpu.SemaphoreType.DMA,
    pltpu.SemaphoreType.DMA,
]


@functools.partial(
    pl.kernel,
    out_type=(_f32((NP, D)), _f32((NP, D)), _f32((NP, D)), _f32((NP, D)),
              _f32((NP, D)), _f32((NP, D))),
    mesh=_mesh,
    compiler_params=pltpu.CompilerParams(use_tc_tiling_on_sc=False),
    scratch_types=_LAYER_SCRATCH,
)
def _layer1_kernel(ue_p, ie_p, w_u, w_i, tu2, ti2,
                   ue1, ie1, tU, tI, a0, b0,
                   sA, sB, dA, dB, rA, rB, wbuf, rbuf, acc,
                   gsA, gsB, ssA, ssB):
    c = lax.axis_index("c")
    s = lax.axis_index("s")
    row0 = s * SPAN
    row_lo = s * RPT

    _zero_rows(rbuf, RCH)

    def zk(k, carry):
        pltpu.sync_copy(rbuf, acc.at[pl.ds(row0 + k * RCH, RCH), :])
        return carry

    lax.fori_loop(0, NRCH, zk, 0)

    def scale_pass(src, dst):
        def sk(k, carry):
            sl = pl.ds(row0 + k * RCH, RCH)
            pltpu.sync_copy(src.at[sl, :], rbuf)
            _scale_rows(rbuf, wbuf, k * RCH)
            pltpu.sync_copy(rbuf, dst.at[sl, :])
            return carry

        lax.fori_loop(0, NRCH, sk, 0)

    @pl.when(c == 0)
    def _():
        pltpu.sync_copy(w_i.at[pl.ds(row0, SPAN)], wbuf)
        scale_pass(ie_p, a0)

    @pl.when(c == 1)
    def _():
        pltpu.sync_copy(w_u.at[pl.ds(row0, SPAN)], wbuf)
        scale_pass(ue_p, b0)

    plsc.subcore_barrier()

    @pl.when(c == 0)
    def _():
        _edge_pipeline(a0, acc, ti2, tu2, sA, sB, dA, dB, rA, rB,
                       gsA, gsB, ssA, ssB, row_lo)

    @pl.when(c == 1)
    def _():
        _edge_pipeline(b0, acc, tu2, ti2, sA, sB, dA, dB, rA, rB,
                       gsA, gsB, ssA, ssB, row_lo)

    plsc.subcore_barrier()

    @pl.when(c == 0)
    def _():
        pltpu.sync_copy(w_u.at[pl.ds(row0, SPAN)], wbuf)

    @pl.when(c == 1)
    def _():
        pltpu.sync_copy(w_i.at[pl.ds(row0, SPAN)], wbuf)

    def drain(eout, tout):
        def dk(k, carry):
            sl = pl.ds(row0 + k * RCH, RCH)
            pltpu.sync_copy(acc.at[sl, :], rbuf)
            _scale_rows(rbuf, wbuf, k * RCH)
            pltpu.sync_copy(rbuf, eout.at[sl, :])
            _scale_rows(rbuf, wbuf, k * RCH)
            pltpu.sync_copy(rbuf, tout.at[sl, :])
            return carry

        lax.fori_loop(0, NRCH, dk, 0)

    @pl.when(c == 0)
    def _():
        drain(ue1, tU)

    @pl.when(c == 1)
    def _():
        drain(ie1, tI)


@functools.partial(
    pl.kernel,
    out_type=(_f32((NP, D)), _f32((NP, D))),
    mesh=_mesh,
    compiler_params=pltpu.CompilerParams(use_tc_tiling_on_sc=False),
    scratch_types=_LAYER_SCRATCH[:7] + [
        pltpu.VMEM((R2, D), jnp.float32),
        _LAYER_SCRATCH[8],
    ] + _LAYER_SCRATCH[9:],
)
def _layer2_kernel(tI, tU, ue_p, ie_p, ue1, ie1, w_u, w_i, tu2, ti2,
                   out_u, out_i,
                   sA, sB, dA, dB, rA, rB, wbuf, rbuf, acc,
                   gsA, gsB, ssA, ssB):
    c = lax.axis_index("c")
    s = lax.axis_index("s")
    row0 = s * SPAN
    row_lo = s * RPT

    _zero_rows(rbuf, R2)

    def zk(k, carry):
        pltpu.sync_copy(rbuf, acc.at[pl.ds(row0 + k * R2, R2), :])
        return carry

    lax.fori_loop(0, NRCH2, zk, 0)

    @pl.when(c == 0)
    def _():
        pltpu.sync_copy(w_u.at[pl.ds(row0, SPAN)], wbuf)

    @pl.when(c == 1)
    def _():
        pltpu.sync_copy(w_i.at[pl.ds(row0, SPAN)], wbuf)

    plsc.subcore_barrier()

    @pl.when(c == 0)
    def _():
        _edge_pipeline(tI, acc, ti2, tu2, sA, sB, dA, dB, rA, rB,
                       gsA, gsB, ssA, ssB, row_lo)

    @pl.when(c == 1)
    def _():
        _edge_pipeline(tU, acc, tu2, ti2, sA, sB, dA, dB, rA, rB,
                       gsA, gsB, ssA, ssB, row_lo)

    plsc.subcore_barrier()

    third = jnp.float32(1.0 / 3.0)

    def drain(e0, e1, out):
        def dk(k, carry):
            sl = pl.ds(row0 + k * R2, R2)
            pltpu.sync_copy(acc.at[sl, :], rbuf)
            pltpu.sync_copy(e0.at[sl, :], rA.at[0, pl.ds(0, R2), :])
            pltpu.sync_copy(e1.at[sl, :], rB.at[0, pl.ds(0, R2), :])
            koff = k * R2

            def st(t, inner):
                wvec = wbuf[pl.ds(koff + t * H, H)]
                for ri in range(H):
                    r = t * H + ri
                    w = wvec[ri]
                    lo = pl.ds(0, H)
                    hi = pl.ds(H, H)
                    rbuf[r, lo] = (rA[0, r, lo] + rB[0, r, lo]
                                   + rbuf[r, lo] * w) * third
                    rbuf[r, hi] = (rA[0, r, hi] + rB[0, r, hi]
                                   + rbuf[r, hi] * w) * third
                return inner

            lax.fori_loop(0, R2 // H, st, 0)
            pltpu.sync_copy(rbuf, out.at[sl, :])
            return carry

        lax.fori_loop(0, NRCH2, dk, 0)

    @pl.when(c == 0)
    def _():
        drain(ue_p, ue1, out_u)

    @pl.when(c == 1)
    def _():
        drain(ie_p, ie1, out_i)


def kernel(user_emb, item_emb, train_user, train_item):
    tu2 = train_user.reshape(ROWS_TOT, SL)
    ti2 = train_item.reshape(ROWS_TOT, SL)
    ones = jnp.ones((SL,), jnp.float32)

    deg_u, deg_i = _deg_kernel(tu2, ti2, ones)
    w_u = lax.rsqrt(jnp.clip(deg_u, 1.0, None))
    w_i = lax.rsqrt(jnp.clip(deg_i, 1.0, None))

    ue_p = jnp.zeros((NP, D), jnp.float32).at[:N_U].set(user_emb)
    ie_p = jnp.zeros((NP, D), jnp.float32).at[:N_I].set(item_emb)

    ue1, ie1, tU, tI, _a0, _b0 = _layer1_kernel(ue_p, ie_p, w_u, w_i, tu2, ti2)
    out_u, out_i = _layer2_kernel(tI, tU, ue_p, ie_p, ue1, ie1,
                                  w_u, w_i, tu2, ti2)
    return jnp.concatenate([out_u[:N_U], out_i[:N_I]], axis=0)


# speedup vs baseline: 52.0705x; 1.5597x over previous
import functools

import jax
import jax.numpy as jnp
from jax import lax
from jax.experimental import pallas as pl
from jax.experimental.pallas import tpu as pltpu
from jax.experimental.pallas import tpu_sc as plsc

N_U = 50000
N_I = 50000
NP = 51200
D = 32
E = 1600000
NS = 16
SPAN = NP // NS
RCH = 80
NRCH = SPAN // RCH
H = 16
SL = 125
ROWS_TOT = E // SL
RPT = ROWS_TOT // NS
NB = 2
SB = 16
NSUP = RPT // SB
NB_D = 8

_mesh = plsc.VectorSubcoreMesh(
    core_axis_name="c", subcore_axis_name="s", num_cores=2, num_subcores=NS)


def _f32(shape):
    return jax.ShapeDtypeStruct(shape, jnp.float32)


def _zero_rows(ref3, nrows):
    z = jnp.zeros((H,), jnp.float32)

    def st(r, carry):
        ref3[0, r, pl.ds(0, H)] = z
        ref3[0, r, pl.ds(H, H)] = z
        return carry

    lax.fori_loop(0, nrows, st, 0)


def _scale_rows(ref3, wbuf, koff, nrows):
    def st(t, carry):
        wvec = wbuf[pl.ds(koff + t * H, H)]
        for ri in range(H):
            r = t * H + ri
            w = wvec[ri]
            ref3[0, r, pl.ds(0, H)] = ref3[0, r, pl.ds(0, H)] * w
            ref3[0, r, pl.ds(H, H)] = ref3[0, r, pl.ds(H, H)] * w
        return carry

    lax.fori_loop(0, nrows // H, st, 0)


def _edge_pipeline(tab, acc, sidx_hbm, didx_hbm, S0s, S1s, S0d, S1d,
                   rA, rB, gsA, gsB, ssA, ssB, isem, row_lo):
    def prefetch(s_idx, Ss, Sd):
        r0 = row_lo + s_idx * SB
        pltpu.async_copy(sidx_hbm.at[pl.ds(r0, SB), :], Ss, isem)
        pltpu.async_copy(didx_hbm.at[pl.ds(r0, SB), :], Sd, isem)

    def wait_prefetch(Ss, Sd):
        pltpu.make_async_copy(sidx_hbm.at[pl.ds(row_lo, SB), :], Ss,
                              isem).wait()
        pltpu.make_async_copy(didx_hbm.at[pl.ds(row_lo, SB), :], Sd,
                              isem).wait()

    def fire_g(Ss, b, rows, gsem):
        for j in range(NB):
            pltpu.async_copy(tab.at[Ss.at[b + j]], rows.at[j], gsem)

    def wait_g(Ss, b, rows, gsem):
        for j in range(NB):
            pltpu.make_async_copy(tab.at[Ss.at[b + j]], rows.at[j],
                                  gsem).wait()

    def fire_s(Sd, b, rows, ssem):
        for j in range(NB):
            pltpu.async_copy(rows.at[j], acc.at[Sd.at[b + j]], ssem, add=True)

    def wait_s(Sd, b, rows, ssem):
        for j in range(NB):
            pltpu.make_async_copy(rows.at[j], acc.at[Sd.at[b + j]],
                                  ssem).wait()

    def superblock(s_idx, Ss, Sd, So_s, So_d):
        wait_prefetch(Ss, Sd)
        for m in range(SB // (2 * NB)):
            b0 = 2 * NB * m

            if m == 0:
                @pl.when(s_idx >= 1)
                def _():
                    wait_s(Sd, b0, rA, ssA)
            else:
                wait_s(Sd, b0, rA, ssA)

            fire_g(Ss, b0, rA, gsA)

            if m == 0:
                @pl.when(s_idx >= 1)
                def _():
                    wait_g(So_s, SB - NB, rB, gsB)
                    fire_s(So_d, SB - NB, rB, ssB)
                    wait_s(So_d, SB - NB, rB, ssB)

                @pl.when(s_idx + 1 < NSUP)
                def _():
                    prefetch(s_idx + 1, So_s, So_d)
            else:
                wait_g(Ss, b0 - NB, rB, gsB)
                fire_s(Sd, b0 - NB, rB, ssB)
                wait_s(Sd, b0 - NB, rB, ssB)

            fire_g(Ss, b0 + NB, rB, gsB)

            wait_g(Ss, b0, rA, gsA)
            fire_s(Sd, b0, rA, ssA)

    prefetch(0, S0s, S0d)

    def outer(o, carry):
        superblock(2 * o, S0s, S0d, S1s, S1d)
        superblock(2 * o + 1, S1s, S1d, S0s, S0d)
        return carry

    lax.fori_loop(0, NSUP // 2, outer, 0)
    wait_g(S1s, SB - NB, rB, gsB)
    fire_s(S1d, SB - NB, rB, ssB)
    wait_s(S1d, 0, rA, ssA)
    wait_s(S1d, 0, rB, ssB)


def _deg_pipeline(acc1, didx_hbm, S0, S1, ones_v, ssA, ssB, isem, row_lo):
    def prefetch(s_idx, Sd):
        r0 = row_lo + s_idx * SB
        pltpu.async_copy(didx_hbm.at[pl.ds(r0, SB), :], Sd, isem)

    def wait_prefetch(Sd):
        pltpu.make_async_copy(didx_hbm.at[pl.ds(row_lo, SB), :], Sd,
                              isem).wait()

    def fire_s(Sd, b, ssem):
        for j in range(NB_D):
            pltpu.async_copy(ones_v, acc1.at[Sd.at[b + j]], ssem, add=True)

    def wait_s(Sd, b, ssem):
        for j in range(NB_D):
            pltpu.make_async_copy(ones_v, acc1.at[Sd.at[b + j]], ssem).wait()

    def superblock(s_idx, Sd, So):
        wait_prefetch(Sd)

        @pl.when(s_idx >= 1)
        def _():
            wait_s(Sd, 0, ssA)

        fire_s(Sd, 0, ssA)

        @pl.when(s_idx >= 1)
        def _():
            wait_s(Sd, 0, ssB)

        @pl.when(s_idx + 1 < NSUP)
        def _():
            prefetch(s_idx + 1, So)

        fire_s(Sd, NB_D, ssB)

    prefetch(0, S0)

    def outer(o, carry):
        superblock(2 * o, S0, S1)
        superblock(2 * o + 1, S1, S0)
        return carry

    lax.fori_loop(0, NSUP // 2, outer, 0)
    wait_s(S0, 0, ssA)
    wait_s(S0, 0, ssB)


@functools.partial(
    pl.kernel,
    out_type=(_f32((NP,)), _f32((NP,))),
    mesh=_mesh,
    compiler_params=pltpu.CompilerParams(use_tc_tiling_on_sc=False),
    scratch_types=[
        pltpu.VMEM((SB, SL), jnp.int32),
        pltpu.VMEM((SB, SL), jnp.int32),
        pltpu.VMEM((SL,), jnp.float32),
        pltpu.VMEM((SPAN,), jnp.float32),
        pltpu.VMEM_SHARED((NP,), jnp.float32),
        pltpu.SemaphoreType.DMA,
        pltpu.SemaphoreType.DMA,
        pltpu.SemaphoreType.DMA,
    ],
)
def _deg_kernel(tu_hbm, ti_hbm, ones_hbm, degu, degi,
                S0, S1, ones_v, sbuf, acc1, ssA, ssB, isem):
    c = lax.axis_index("c")
    s = lax.axis_index("s")
    row0 = s * SPAN

    z = jnp.zeros((H,), jnp.float32)

    def zst(r, carry):
        sbuf[pl.ds(r * H, H)] = z
        return carry

    lax.fori_loop(0, SPAN // H, zst, 0)
    pltpu.sync_copy(sbuf, acc1.at[pl.ds(row0, SPAN)])
    pltpu.sync_copy(ones_hbm, ones_v)
    plsc.subcore_barrier()

    row_lo = s * RPT

    @pl.when(c == 0)
    def _():
        _deg_pipeline(acc1, tu_hbm, S0, S1, ones_v, ssA, ssB, isem, row_lo)

    @pl.when(c == 1)
    def _():
        _deg_pipeline(acc1, ti_hbm, S0, S1, ones_v, ssA, ssB, isem, row_lo)

    plsc.subcore_barrier()

    pltpu.sync_copy(acc1.at[pl.ds(row0, SPAN)], sbuf)

    @pl.when(c == 0)
    def _():
        pltpu.sync_copy(sbuf, degu.at[pl.ds(row0, SPAN)])

    @pl.when(c == 1)
    def _():
        pltpu.sync_copy(sbuf, degi.at[pl.ds(row0, SPAN)])


_LAYER_SCRATCH = [
    pltpu.VMEM((SB, SL), jnp.int32),
    pltpu.VMEM((SB, SL), jnp.int32),
    pltpu.VMEM((SB, SL), jnp.int32),
    pltpu.VMEM((SB, SL), jnp.int32),
    pltpu.VMEM((NB, SL, D), jnp.float32),
    pltpu.VMEM((NB, SL, D), jnp.float32),
    pltpu.VMEM((SPAN,), jnp.float32),
    pltpu.VMEM_SHARED((NP, D), jnp.float32),
    pltpu.SemaphoreType.DMA,
    pltpu.SemaphoreType.DMA,
    pltpu.SemaphoreType.DMA,
    pltpu.SemaphoreType.DMA,
    pltpu.SemaphoreType.DMA,
]


@functools.partial(
    pl.kernel,
    out_type=(_f32((NP, D)), _f32((NP, D)), _f32((NP, D)), _f32((NP, D)),
              _f32((NP, D)), _f32((NP, D))),
    mesh=_mesh,
    compiler_params=pltpu.CompilerParams(use_tc_tiling_on_sc=False),
    scratch_types=_LAYER_SCRATCH,
)
def _layer1_kernel(ue_p, ie_p, w_u, w_i, tu2, ti2,
                   ue1, ie1, tU, tI, a0, b0,
                   S0s, S1s, S0d, S1d, rA, rB, wbuf, acc,
                   gsA, gsB, ssA, ssB, isem):
    c = lax.axis_index("c")
    s = lax.axis_index("s")
    row0 = s * SPAN
    row_lo = s * RPT
    rb = rA.at[0, pl.ds(0, RCH), :]

    _zero_rows(rA, RCH)

    def zk(k, carry):
        pltpu.sync_copy(rb, acc.at[pl.ds(row0 + k * RCH, RCH), :])
        return carry

    lax.fori_loop(0, NRCH, zk, 0)

    def scale_pass(src, dst):
        def sk(k, carry):
            sl = pl.ds(row0 + k * RCH, RCH)
            pltpu.sync_copy(src.at[sl, :], rb)
            _scale_rows(rA, wbuf, k * RCH, RCH)
            pltpu.sync_copy(rb, dst.at[sl, :])
            return carry

        lax.fori_loop(0, NRCH, sk, 0)

    @pl.when(c == 0)
    def _():
        pltpu.sync_copy(w_i.at[pl.ds(row0, SPAN)], wbuf)
        scale_pass(ie_p, a0)

    @pl.when(c == 1)
    def _():
        pltpu.sync_copy(w_u.at[pl.ds(row0, SPAN)], wbuf)
        scale_pass(ue_p, b0)

    plsc.subcore_barrier()

    @pl.when(c == 0)
    def _():
        _edge_pipeline(a0, acc, ti2, tu2, S0s, S1s, S0d, S1d, rA, rB,
                       gsA, gsB, ssA, ssB, isem, row_lo)

    @pl.when(c == 1)
    def _():
        _edge_pipeline(b0, acc, tu2, ti2, S0s, S1s, S0d, S1d, rA, rB,
                       gsA, gsB, ssA, ssB, isem, row_lo)

    plsc.subcore_barrier()

    @pl.when(c == 0)
    def _():
        pltpu.sync_copy(w_u.at[pl.ds(row0, SPAN)], wbuf)

    @pl.when(c == 1)
    def _():
        pltpu.sync_copy(w_i.at[pl.ds(row0, SPAN)], wbuf)

    def drain(eout, tout):
        def dk(k, carry):
            sl = pl.ds(row0 + k * RCH, RCH)
            pltpu.sync_copy(acc.at[sl, :], rb)
            _scale_rows(rA, wbuf, k * RCH, RCH)
            pltpu.sync_copy(rb, eout.at[sl, :])
            _scale_rows(rA, wbuf, k * RCH, RCH)
            pltpu.sync_copy(rb, tout.at[sl, :])
            return carry

        lax.fori_loop(0, NRCH, dk, 0)

    @pl.when(c == 0)
    def _():
        drain(ue1, tU)

    @pl.when(c == 1)
    def _():
        drain(ie1, tI)


@functools.partial(
    pl.kernel,
    out_type=(_f32((NP, D)), _f32((NP, D))),
    mesh=_mesh,
    compiler_params=pltpu.CompilerParams(use_tc_tiling_on_sc=False),
    scratch_types=_LAYER_SCRATCH,
)
def _layer2_kernel(tI, tU, ue_p, ie_p, ue1, ie1, w_u, w_i, tu2, ti2,
                   out_u, out_i,
                   S0s, S1s, S0d, S1d, rA, rB, wbuf, acc,
                   gsA, gsB, ssA, ssB, isem):
    c = lax.axis_index("c")
    s = lax.axis_index("s")
    row0 = s * SPAN
    row_lo = s * RPT
    rb = rA.at[0, pl.ds(0, RCH), :]

    _zero_rows(rA, RCH)

    def zk(k, carry):
        pltpu.sync_copy(rb, acc.at[pl.ds(row0 + k * RCH, RCH), :])
        return carry

    lax.fori_loop(0, NRCH, zk, 0)

    @pl.when(c == 0)
    def _():
        pltpu.sync_copy(w_u.at[pl.ds(row0, SPAN)], wbuf)

    @pl.when(c == 1)
    def _():
        pltpu.sync_copy(w_i.at[pl.ds(row0, SPAN)], wbuf)

    plsc.subcore_barrier()

    @pl.when(c == 0)
    def _():
        _edge_pipeline(tI, acc, ti2, tu2, S0s, S1s, S0d, S1d, rA, rB,
                       gsA, gsB, ssA, ssB, isem, row_lo)

    @pl.when(c == 1)
    def _():
        _edge_pipeline(tU, acc, tu2, ti2, S0s, S1s, S0d, S1d, rA, rB,
                       gsA, gsB, ssA, ssB, isem, row_lo)

    plsc.subcore_barrier()

    third = jnp.float32(1.0 / 3.0)

    def drain(e0, e1, out):
        def dk(k, carry):
            sl = pl.ds(row0 + k * RCH, RCH)
            pltpu.sync_copy(acc.at[sl, :], rb)
            pltpu.sync_copy(e0.at[sl, :], rA.at[1, pl.ds(0, RCH), :])
            pltpu.sync_copy(e1.at[sl, :], rB.at[0, pl.ds(0, RCH), :])
            koff = k * RCH

            def st(t, inner):
                wvec = wbuf[pl.ds(koff + t * H, H)]
                for ri in range(H):
                    r = t * H + ri
                    w = wvec[ri]
                    lo = pl.ds(0, H)
                    hi = pl.ds(H, H)
                    rA[0, r, lo] = (rA[1, r, lo] + rB[0, r, lo]
                                    + rA[0, r, lo] * w) * third
                    rA[0, r, hi] = (rA[1, r, hi] + rB[0, r, hi]
                                    + rA[0, r, hi] * w) * third
                return inner

            lax.fori_loop(0, RCH // H, st, 0)
            pltpu.sync_copy(rb, out.at[sl, :])
            return carry

        lax.fori_loop(0, NRCH, dk, 0)

    @pl.when(c == 0)
    def _():
        drain(ue_p, ue1, out_u)

    @pl.when(c == 1)
    def _():
        drain(ie_p, ie1, out_i)


def kernel(user_emb, item_emb, train_user, train_item):
    tu2 = train_user.reshape(ROWS_TOT, SL)
    ti2 = train_item.reshape(ROWS_TOT, SL)
    ones = jnp.ones((SL,), jnp.float32)

    deg_u, deg_i = _deg_kernel(tu2, ti2, ones)
    w_u = lax.rsqrt(jnp.clip(deg_u, 1.0, None))
    w_i = lax.rsqrt(jnp.clip(deg_i, 1.0, None))

    ue_p = jnp.zeros((NP, D), jnp.float32).at[:N_U].set(user_emb)
    ie_p = jnp.zeros((NP, D), jnp.float32).at[:N_I].set(item_emb)

    ue1, ie1, tU, tI, _a0, _b0 = _layer1_kernel(ue_p, ie_p, w_u, w_i, tu2, ti2)
    out_u, out_i = _layer2_kernel(tI, tU, ue_p, ie_p, ue1, ie1,
                                  w_u, w_i, tu2, ti2)
    return jnp.concatenate([out_u[:N_U], out_i[:N_I]], axis=0)


# speedup vs baseline: 54.4177x; 1.0451x over previous
import functools

import jax
import jax.numpy as jnp
from jax import lax
from jax.experimental import pallas as pl
from jax.experimental.pallas import tpu as pltpu
from jax.experimental.pallas import tpu_sc as plsc

N_U = 50000
N_I = 50000
NP = 51200
D = 32
E = 1600000
NS = 16
SPAN = NP // NS
H = 16
SL = 125
ROWS_TOT = E // SL
RPT = ROWS_TOT // NS
NB = 2
SB = 16
NSUP = RPT // SB
NB_D = 8
RROWS = NB * SL
CH = 160
NCH = SPAN // CH
CH2 = 80
NCH2 = SPAN // CH2
CH1 = 80
NCH1 = SPAN // CH1

_mesh = plsc.VectorSubcoreMesh(
    core_axis_name="c", subcore_axis_name="s", num_cores=2, num_subcores=NS)


def _f32(shape):
    return jax.ShapeDtypeStruct(shape, jnp.float32)


def _rsqrt16(x):
    magic = jnp.full((H,), 0x5F3759DF, jnp.int32)
    one_i = jnp.full((H,), 1, jnp.int32)
    c15 = jnp.full((H,), 1.5, jnp.float32)
    c05 = jnp.full((H,), 0.5, jnp.float32)
    i = lax.bitcast_convert_type(x, jnp.int32)
    i = magic - jnp.right_shift(i, one_i)
    y = lax.bitcast_convert_type(i, jnp.float32)
    for _ in range(4):
        y = y * (c15 - c05 * x * y * y)
    return y


def _edge_pipeline(tab, acc, sidx_hbm, didx_hbm, S0s, S1s, S0d, S1d,
                   rA, rB, gsA, gsB, ssA, ssB, isem, row_lo):
    def slot(buf, j):
        return buf.at[pl.ds(j * SL, SL), :]

    def prefetch(s_idx, Ss, Sd):
        r0 = row_lo + s_idx * SB
        pltpu.async_copy(sidx_hbm.at[pl.ds(r0, SB), :], Ss, isem)
        pltpu.async_copy(didx_hbm.at[pl.ds(r0, SB), :], Sd, isem)

    def wait_prefetch(Ss, Sd):
        pltpu.make_async_copy(sidx_hbm.at[pl.ds(row_lo, SB), :], Ss,
                              isem).wait()
        pltpu.make_async_copy(didx_hbm.at[pl.ds(row_lo, SB), :], Sd,
                              isem).wait()

    def fire_g(Ss, b, rows, gsem):
        for j in range(NB):
            pltpu.async_copy(tab.at[Ss.at[b + j]], slot(rows, j), gsem)

    def wait_g(Ss, b, rows, gsem):
        for j in range(NB):
            pltpu.make_async_copy(tab.at[Ss.at[b + j]], slot(rows, j),
                                  gsem).wait()

    def fire_s(Sd, b, rows, ssem):
        for j in range(NB):
            pltpu.async_copy(slot(rows, j), acc.at[Sd.at[b + j]], ssem,
                             add=True)

    def wait_s(Sd, b, rows, ssem):
        for j in range(NB):
            pltpu.make_async_copy(slot(rows, j), acc.at[Sd.at[b + j]],
                                  ssem).wait()

    def superblock(s_idx, Ss, Sd, So_s, So_d):
        wait_prefetch(Ss, Sd)
        for m in range(SB // (2 * NB)):
            b0 = 2 * NB * m

            if m == 0:
                @pl.when(s_idx >= 1)
                def _():
                    wait_s(Sd, b0, rA, ssA)
            else:
                wait_s(Sd, b0, rA, ssA)

            fire_g(Ss, b0, rA, gsA)

            if m == 0:
                @pl.when(s_idx >= 1)
                def _():
                    wait_g(So_s, SB - NB, rB, gsB)
                    fire_s(So_d, SB - NB, rB, ssB)
                    wait_s(So_d, SB - NB, rB, ssB)

                @pl.when(s_idx + 1 < NSUP)
                def _():
                    prefetch(s_idx + 1, So_s, So_d)
            else:
                wait_g(Ss, b0 - NB, rB, gsB)
                fire_s(Sd, b0 - NB, rB, ssB)
                wait_s(Sd, b0 - NB, rB, ssB)

            fire_g(Ss, b0 + NB, rB, gsB)

            wait_g(Ss, b0, rA, gsA)
            fire_s(Sd, b0, rA, ssA)

    prefetch(0, S0s, S0d)

    def outer(o, carry):
        superblock(2 * o, S0s, S0d, S1s, S1d)
        superblock(2 * o + 1, S1s, S1d, S0s, S0d)
        return carry

    lax.fori_loop(0, NSUP // 2, outer, 0)
    wait_g(S1s, SB - NB, rB, gsB)
    fire_s(S1d, SB - NB, rB, ssB)
    wait_s(S1d, 0, rA, ssA)
    wait_s(S1d, 0, rB, ssB)


def _deg_pipeline(acc1, didx_hbm, S0, S1, ones_v, ssA, ssB, isem, row_lo):
    def prefetch(s_idx, Sd):
        r0 = row_lo + s_idx * SB
        pltpu.async_copy(didx_hbm.at[pl.ds(r0, SB), :], Sd, isem)

    def wait_prefetch(Sd):
        pltpu.make_async_copy(didx_hbm.at[pl.ds(row_lo, SB), :], Sd,
                              isem).wait()

    def fire_s(Sd, b, ssem):
        for j in range(NB_D):
            pltpu.async_copy(ones_v, acc1.at[Sd.at[b + j]], ssem, add=True)

    def wait_s(Sd, b, ssem):
        for j in range(NB_D):
            pltpu.make_async_copy(ones_v, acc1.at[Sd.at[b + j]], ssem).wait()

    def superblock(s_idx, Sd, So):
        wait_prefetch(Sd)

        @pl.when(s_idx >= 1)
        def _():
            wait_s(Sd, 0, ssA)

        fire_s(Sd, 0, ssA)

        @pl.when(s_idx >= 1)
        def _():
            wait_s(Sd, 0, ssB)

        @pl.when(s_idx + 1 < NSUP)
        def _():
            prefetch(s_idx + 1, So)

        fire_s(Sd, NB_D, ssB)

    prefetch(0, S0)

    def outer(o, carry):
        superblock(2 * o, S0, S1)
        superblock(2 * o + 1, S1, S0)
        return carry

    lax.fori_loop(0, NSUP // 2, outer, 0)
    wait_s(S0, 0, ssA)
    wait_s(S0, 0, ssB)


def _scale_chunk(buf, wbuf, koff, nrows, square):
    def st(t, carry):
        wvec = wbuf[pl.ds(koff + t * H, H)]
        if square:
            wvec = wvec * wvec
        for ri in range(H):
            r = t * H + ri
            w = wvec[ri]
            buf[r, pl.ds(0, H)] = buf[r, pl.ds(0, H)] * w
            buf[r, pl.ds(H, H)] = buf[r, pl.ds(H, H)] * w
        return carry

    lax.fori_loop(0, nrows // H, st, 0)


def _scale_chunk_to(buf, wbuf, koff, nrows):
    def st(t, carry):
        wvec = wbuf[pl.ds(koff + t * H, H)]
        wvec = wvec * wvec
        for ri in range(H):
            r = t * H + ri
            w = wvec[ri]
            buf[CH1 + r, pl.ds(0, H)] = buf[r, pl.ds(0, H)] * w
            buf[CH1 + r, pl.ds(H, H)] = buf[r, pl.ds(H, H)] * w
        return carry

    lax.fori_loop(0, nrows // H, st, 0)


def _scale_stream(src, dst, wbuf, row0, bufA, bufB, inA, inB, outA, outB,
                  square):
    def sl(k):
        return pl.ds(row0 + k * CH, CH)

    bA = bufA.at[pl.ds(0, CH), :]
    bB = bufB.at[pl.ds(0, CH), :]

    pltpu.async_copy(src.at[sl(0)], bA, inA)

    def pair(p, carry):
        kA = 2 * p
        kB = 2 * p + 1
        pltpu.make_async_copy(src.at[sl(0)], bA, inA).wait()

        @pl.when(p >= 1)
        def _():
            pltpu.make_async_copy(bB, dst.at[sl(0)], outB).wait()

        pltpu.async_copy(src.at[sl(kB)], bB, inB)
        _scale_chunk(bufA, wbuf, kA * CH, CH, square)
        pltpu.async_copy(bA, dst.at[sl(kA)], outA)
        pltpu.make_async_copy(src.at[sl(0)], bB, inB).wait()
        _scale_chunk(bufB, wbuf, kB * CH, CH, square)
        pltpu.make_async_copy(bA, dst.at[sl(0)], outA).wait()

        @pl.when(kA + 2 < NCH)
        def _():
            pltpu.async_copy(src.at[sl(kA + 2)], bA, inA)

        pltpu.async_copy(bB, dst.at[sl(kB)], outB)
        return carry

    lax.fori_loop(0, NCH // 2, pair, 0)
    pltpu.make_async_copy(bB, dst.at[pl.ds(row0, CH), :], outB).wait()


def _zero_acc(acc2, row0, buf, sem):
    z = jnp.zeros((H,), jnp.float32)

    def zst(r, carry):
        buf[r, pl.ds(0, H)] = z
        buf[r, pl.ds(H, H)] = z
        return carry

    lax.fori_loop(0, CH, zst, 0)
    bz = buf.at[pl.ds(0, CH), :]

    def zk(k, carry):
        pltpu.async_copy(bz, acc2.at[pl.ds(row0 + k * CH, CH), :], sem)
        return carry

    lax.fori_loop(0, NCH, zk, 0)

    def zw(k, carry):
        pltpu.make_async_copy(bz, acc2.at[pl.ds(row0, CH), :], sem).wait()
        return carry

    lax.fori_loop(0, NCH, zw, 0)


@functools.partial(
    pl.kernel,
    out_type=(_f32((NP,)), _f32((NP,)), _f32((NP, D)), _f32((NP, D))),
    mesh=_mesh,
    compiler_params=pltpu.CompilerParams(use_tc_tiling_on_sc=False),
    scratch_types=[
        pltpu.VMEM((SB, SL), jnp.int32),
        pltpu.VMEM((SB, SL), jnp.int32),
        pltpu.VMEM((SL,), jnp.float32),
        pltpu.VMEM((SPAN,), jnp.float32),
        pltpu.VMEM((SPAN,), jnp.float32),
        pltpu.VMEM((RROWS, D), jnp.float32),
        pltpu.VMEM((RROWS, D), jnp.float32),
        pltpu.VMEM_SHARED((NP,), jnp.float32),
        pltpu.SemaphoreType.DMA,
        pltpu.SemaphoreType.DMA,
        pltpu.SemaphoreType.DMA,
        pltpu.SemaphoreType.DMA,
    ],
)
def _deg_kernel(tu_hbm, ti_hbm, ones_hbm, ue_p, ie_p,
                w_u, w_i, b0, a0,
                S0, S1, ones_v, sbuf, wbuf, rAf, rBf, acc1,
                ssA, ssB, isem, osem):
    c = lax.axis_index("c")
    s = lax.axis_index("s")
    row0 = s * SPAN

    z = jnp.zeros((H,), jnp.float32)

    def zst(r, carry):
        sbuf[pl.ds(r * H, H)] = z
        return carry

    lax.fori_loop(0, SPAN // H, zst, 0)
    pltpu.sync_copy(sbuf, acc1.at[pl.ds(row0, SPAN)])
    pltpu.sync_copy(ones_hbm, ones_v)
    plsc.subcore_barrier()

    row_lo = s * RPT

    @pl.when(c == 0)
    def _():
        _deg_pipeline(acc1, tu_hbm, S0, S1, ones_v, ssA, ssB, isem, row_lo)

    @pl.when(c == 1)
    def _():
        _deg_pipeline(acc1, ti_hbm, S0, S1, ones_v, ssA, ssB, isem, row_lo)

    plsc.subcore_barrier()

    pltpu.sync_copy(acc1.at[pl.ds(row0, SPAN)], sbuf)

    def wg(t, carry):
        x = jnp.maximum(sbuf[pl.ds(t * H, H)],
                        jnp.full((H,), 1.0, jnp.float32))
        wbuf[pl.ds(t * H, H)] = _rsqrt16(x)
        return carry

    lax.fori_loop(0, SPAN // H, wg, 0)

    @pl.when(c == 0)
    def _():
        pltpu.sync_copy(wbuf, w_u.at[pl.ds(row0, SPAN)])
        _scale_stream(ue_p, b0, wbuf, row0, rAf, rBf,
                      ssA, ssB, isem, osem, square=False)

    @pl.when(c == 1)
    def _():
        pltpu.sync_copy(wbuf, w_i.at[pl.ds(row0, SPAN)])
        _scale_stream(ie_p, a0, wbuf, row0, rAf, rBf,
                      ssA, ssB, isem, osem, square=False)


_LAYER_SCRATCH = [
    pltpu.VMEM((SB, SL), jnp.int32),
    pltpu.VMEM((SB, SL), jnp.int32),
    pltpu.VMEM((SB, SL), jnp.int32),
    pltpu.VMEM((SB, SL), jnp.int32),
    pltpu.VMEM((RROWS, D), jnp.float32),
    pltpu.VMEM((RROWS, D), jnp.float32),
    pltpu.VMEM((SPAN,), jnp.float32),
    pltpu.VMEM_SHARED((NP, D), jnp.float32),
    pltpu.SemaphoreType.DMA,
    pltpu.SemaphoreType.DMA,
    pltpu.SemaphoreType.DMA,
    pltpu.SemaphoreType.DMA,
    pltpu.SemaphoreType.DMA,
    pltpu.SemaphoreType.DMA,
]


@functools.partial(
    pl.kernel,
    out_type=(_f32((NP, D)), _f32((NP, D)), _f32((NP, D)), _f32((NP, D))),
    mesh=_mesh,
    compiler_params=pltpu.CompilerParams(use_tc_tiling_on_sc=False),
    scratch_types=_LAYER_SCRATCH,
)
def _layer1_kernel(a0, b0, w_u, w_i, tu2, ti2,
                   r1u, r1i, tU, tI,
                   S0s, S1s, S0d, S1d, rAf, rBf, wbuf, acc,
                   gsA, gsB, ssA, ssB, isem, osem):
    c = lax.axis_index("c")
    s = lax.axis_index("s")
    row0 = s * SPAN
    row_lo = s * RPT

    _zero_acc(acc, row0, rAf, isem)
    plsc.subcore_barrier()

    @pl.when(c == 0)
    def _():
        _edge_pipeline(a0, acc, ti2, tu2, S0s, S1s, S0d, S1d, rAf, rBf,
                       gsA, gsB, ssA, ssB, isem, row_lo)

    @pl.when(c == 1)
    def _():
        _edge_pipeline(b0, acc, tu2, ti2, S0s, S1s, S0d, S1d, rAf, rBf,
                       gsA, gsB, ssA, ssB, isem, row_lo)

    plsc.subcore_barrier()

    sp = pl.ds(row0, SPAN)

    def drain(w_hbm, raw_out, t_out):
        pltpu.sync_copy(w_hbm.at[sp], wbuf)

        def sl(k):
            return pl.ds(row0 + k * CH1, CH1)

        i0A = rAf.at[pl.ds(0, CH1), :]
        o1A = rAf.at[pl.ds(CH1, CH1), :]
        i0B = rBf.at[pl.ds(0, CH1), :]
        o1B = rBf.at[pl.ds(CH1, CH1), :]

        pltpu.async_copy(acc.at[sl(0), :], i0A, gsA)

        def pair(p, carry):
            kA = 2 * p
            kB = 2 * p + 1
            pltpu.make_async_copy(acc.at[sl(0), :], i0A, gsA).wait()
            pltpu.async_copy(i0A, raw_out.at[sl(kA), :], isem)

            @pl.when(p >= 1)
            def _():
                pltpu.make_async_copy(o1B, t_out.at[sl(0), :], ssB).wait()
                pltpu.make_async_copy(i0B, raw_out.at[sl(0), :],
                                      osem).wait()

            pltpu.async_copy(acc.at[sl(kB), :], i0B, gsB)
            _scale_chunk_to(rAf, wbuf, kA * CH1, CH1)
            pltpu.async_copy(o1A, t_out.at[sl(kA), :], ssA)
            pltpu.make_async_copy(acc.at[sl(0), :], i0B, gsB).wait()
            pltpu.async_copy(i0B, raw_out.at[sl(kB), :], osem)
            pltpu.make_async_copy(i0A, raw_out.at[sl(0), :], isem).wait()

            @pl.when(kA + 2 < NCH1)
            def _():
                pltpu.async_copy(acc.at[sl(kA + 2), :], i0A, gsA)

            pltpu.make_async_copy(o1A, t_out.at[sl(0), :], ssA).wait()
            _scale_chunk_to(rBf, wbuf, kB * CH1, CH1)
            pltpu.async_copy(o1B, t_out.at[sl(kB), :], ssB)
            return carry

        lax.fori_loop(0, NCH1 // 2, pair, 0)
        pltpu.make_async_copy(i0B, raw_out.at[pl.ds(row0, CH1), :],
                              osem).wait()
        pltpu.make_async_copy(o1B, t_out.at[pl.ds(row0, CH1), :],
                              ssB).wait()

    @pl.when(c == 0)
    def _():
        drain(w_u, r1u, tU)

    @pl.when(c == 1)
    def _():
        drain(w_i, r1i, tI)


@functools.partial(
    pl.kernel,
    out_type=(_f32((NP, D)), _f32((NP, D))),
    mesh=_mesh,
    compiler_params=pltpu.CompilerParams(use_tc_tiling_on_sc=False),
    scratch_types=_LAYER_SCRATCH,
)
def _layer2_kernel(tI, tU, ue_p, ie_p, r1u, r1i, w_u, w_i, tu2, ti2,
                   out_u, out_i,
                   S0s, S1s, S0d, S1d, rAf, rBf, wbuf, acc,
                   gsA, gsB, ssA, ssB, isem, osem):
    c = lax.axis_index("c")
    s = lax.axis_index("s")
    row0 = s * SPAN
    row_lo = s * RPT

    _zero_acc(acc, row0, rAf, isem)

    @pl.when(c == 0)
    def _():
        pltpu.sync_copy(w_u.at[pl.ds(row0, SPAN)], wbuf)

    @pl.when(c == 1)
    def _():
        pltpu.sync_copy(w_i.at[pl.ds(row0, SPAN)], wbuf)

    plsc.subcore_barrier()

    @pl.when(c == 0)
    def _():
        _edge_pipeline(tI, acc, ti2, tu2, S0s, S1s, S0d, S1d, rAf, rBf,
                       gsA, gsB, ssA, ssB, isem, row_lo)

    @pl.when(c == 1)
    def _():
        _edge_pipeline(tU, acc, tu2, ti2, S0s, S1s, S0d, S1d, rAf, rBf,
                       gsA, gsB, ssA, ssB, isem, row_lo)

    plsc.subcore_barrier()

    third = jnp.float32(1.0 / 3.0)

    def mean_chunk(buf, koff):
        def st(t, carry):
            wvec = wbuf[pl.ds(koff + t * H, H)]
            for ri in range(H):
                r = t * H + ri
                w = wvec[ri]
                lo = pl.ds(0, H)
                hi = pl.ds(H, H)
                buf[r, lo] = (buf[2 * CH2 + r, lo]
                              + (buf[CH2 + r, lo] + buf[r, lo]) * w) * third
                buf[r, hi] = (buf[2 * CH2 + r, hi]
                              + (buf[CH2 + r, hi] + buf[r, hi]) * w) * third
            return carry

        lax.fori_loop(0, CH2 // H, st, 0)

    def drain(e0, raw1, out):
        def sl(k):
            return pl.ds(row0 + k * CH2, CH2)

        def fire_in(k, buf, sem):
            pltpu.async_copy(acc.at[sl(k), :], buf.at[pl.ds(0, CH2), :], sem)
            pltpu.async_copy(raw1.at[sl(k), :],
                             buf.at[pl.ds(CH2, CH2), :], isem)
            pltpu.async_copy(e0.at[sl(k), :],
                             buf.at[pl.ds(2 * CH2, CH2), :], osem)

        def wait_in(buf, sem):
            pltpu.make_async_copy(acc.at[pl.ds(row0, CH2), :],
                                  buf.at[pl.ds(0, CH2), :], sem).wait()
            pltpu.make_async_copy(raw1.at[pl.ds(row0, CH2), :],
                                  buf.at[pl.ds(CH2, CH2), :], isem).wait()
            pltpu.make_async_copy(e0.at[pl.ds(row0, CH2), :],
                                  buf.at[pl.ds(2 * CH2, CH2), :],
                                  osem).wait()

        bA = rAf.at[pl.ds(0, CH2), :]
        bB = rBf.at[pl.ds(0, CH2), :]
        fire_in(0, rAf, gsA)

        def pair(p, carry):
            kA = 2 * p
            kB = 2 * p + 1
            wait_in(rAf, gsA)

            @pl.when(p >= 1)
            def _():
                pltpu.make_async_copy(bB, out.at[sl(0), :], ssB).wait()

            fire_in(kB, rBf, gsB)
            mean_chunk(rAf, kA * CH2)
            pltpu.async_copy(bA, out.at[sl(kA), :], ssA)
            wait_in(rBf, gsB)
            mean_chunk(rBf, kB * CH2)
            pltpu.make_async_copy(bA, out.at[sl(0), :], ssA).wait()

            @pl.when(kA + 2 < NCH2)
            def _():
                fire_in(kA + 2, rAf, gsA)

            pltpu.async_copy(bB, out.at[sl(kB), :], ssB)
            return carry

        lax.fori_loop(0, NCH2 // 2, pair, 0)
        pltpu.make_async_copy(bB, out.at[pl.ds(row0, CH2), :], ssB).wait()

    @pl.when(c == 0)
    def _():
        drain(ue_p, r1u, out_u)

    @pl.when(c == 1)
    def _():
        drain(ie_p, r1i, out_i)


def kernel(user_emb, item_emb, train_user, train_item):
    tu2 = train_user.reshape(ROWS_TOT, SL)
    ti2 = train_item.reshape(ROWS_TOT, SL)
    ones = jnp.ones((SL,), jnp.float32)

    ue_p = jnp.zeros((NP, D), jnp.float32).at[:N_U].set(user_emb)
    ie_p = jnp.zeros((NP, D), jnp.float32).at[:N_I].set(item_emb)

    w_u, w_i, b0, a0 = _deg_kernel(tu2, ti2, ones, ue_p, ie_p)
    r1u, r1i, tU, tI = _layer1_kernel(a0, b0, w_u, w_i, tu2, ti2)
    out_u, out_i = _layer2_kernel(tI, tU, ue_p, ie_p, r1u, r1i,
                                  w_u, w_i, tu2, ti2)
    return jnp.concatenate([out_u[:N_U], out_i[:N_I]], axis=0)
